# Initial kernel scaffold; baseline (speedup 1.0000x reference)
import jax
import jax.numpy as jnp
from jax.experimental import pallas as pl


def kernel(t_sampled, batch):
    raise NotImplementedError("write your pallas kernel here")


# speedup vs baseline: 2.3380x; 2.3380x over previous
import functools

import jax
import jax.numpy as jnp
from jax import lax
from jax.experimental import pallas as pl
from jax.experimental.pallas import tpu as pltpu
from jax.experimental.pallas import tpu_sc as plsc

_EMB = 128
_HALF = 64
_NG = 8192
_NA = 524288

_NC = 2
_NS = 16
_NW = _NC * _NS

_CHUNK = 128
_ROWS_PER_W = _NA // _NW
_NCHUNKS = _ROWS_PER_W // _CHUNK


def _emb_body(t_ref, out_ref):
    t = t_ref[:, :]
    k = lax.broadcasted_iota(jnp.int32, (1, _EMB), 1).astype(jnp.float32)
    kmod = jnp.where(k < _HALF, k, k - _HALF)
    freqs = jnp.exp(-jnp.log(10000.0) * kmod / (_HALF - 1))
    phase = jnp.where(k < _HALF, 0.0, jnp.pi / 2.0)
    out_ref[:, :] = jnp.sin(t * freqs + phase)


def _emb_table(t_sampled):
    return pl.pallas_call(
        _emb_body,
        out_shape=jax.ShapeDtypeStruct((_NG, _EMB), jnp.float32),
    )(t_sampled)


def _sc_gather_body(emb_hbm, batch_hbm, out_hbm, idx_v, rows_v, sem):
    wid = lax.axis_index("s") * _NC + lax.axis_index("c")
    base = wid * _ROWS_PER_W
    pltpu.sync_copy(batch_hbm.at[wid], idx_v)

    def chunk(j, carry):
        pltpu.async_copy(emb_hbm.at[idx_v.at[j]], rows_v, sem).wait()
        pltpu.sync_copy(rows_v, out_hbm.at[pl.ds(base + j * _CHUNK, _CHUNK)])
        return carry

    lax.fori_loop(0, _NCHUNKS, chunk, 0)


_sc_gather = functools.partial(
    pl.kernel,
    mesh=plsc.VectorSubcoreMesh(core_axis_name="c", subcore_axis_name="s"),
    out_type=jax.ShapeDtypeStruct((_NA, _EMB), jnp.float32),
    scratch_types=[
        pltpu.VMEM((_NCHUNKS, _CHUNK), jnp.int32),
        pltpu.VMEM((_CHUNK, _EMB), jnp.float32),
        pltpu.SemaphoreType.DMA,
    ],
)(_sc_gather_body)


def kernel(t_sampled, batch):
    emb = _emb_table(t_sampled.astype(jnp.float32))
    batch_r = batch.reshape(_NW, _NCHUNKS, _CHUNK)
    return _sc_gather(emb, batch_r)


# speedup vs baseline: 3.9143x; 1.6742x over previous
import functools

import jax
import jax.numpy as jnp
from jax import lax
from jax.experimental import pallas as pl
from jax.experimental.pallas import tpu as pltpu
from jax.experimental.pallas import tpu_sc as plsc

_EMB = 128
_HALF = 64
_NG = 8192
_NA = 524288

_NC = 2
_NS = 16
_NW = _NC * _NS

_CHUNK = 128
_ROWS_PER_W = _NA // _NW
_NCHUNKS = _ROWS_PER_W // _CHUNK


def _emb_body(t_ref, out_ref):
    t = t_ref[:, :]
    k = lax.broadcasted_iota(jnp.int32, (1, _EMB), 1).astype(jnp.float32)
    kmod = jnp.where(k < _HALF, k, k - _HALF)
    freqs = jnp.exp(-jnp.log(10000.0) * kmod / (_HALF - 1))
    phase = jnp.where(k < _HALF, 0.0, jnp.pi / 2.0)
    out_ref[:, :] = jnp.sin(t * freqs + phase)


def _emb_table(t_sampled):
    return pl.pallas_call(
        _emb_body,
        out_shape=jax.ShapeDtypeStruct((_NG, _EMB), jnp.float32),
    )(t_sampled)


_NBUF = 4


def _sc_gather_body(emb_hbm, batch_hbm, out_hbm, idx_v, rows_v, gsem, ssem):
    wid = lax.axis_index("s") * _NC + lax.axis_index("c")
    base = wid * _ROWS_PER_W
    pltpu.sync_copy(batch_hbm.at[wid], idx_v)

    def start_gather(j, b):
        pltpu.async_copy(emb_hbm.at[idx_v.at[j]], rows_v.at[b], gsem.at[b])

    def wait_gather(b):
        pltpu.make_async_copy(
            emb_hbm.at[pl.ds(0, _CHUNK)], rows_v.at[b], gsem.at[b]
        ).wait()

    def start_scatter(j, b):
        pltpu.async_copy(
            rows_v.at[b], out_hbm.at[pl.ds(base + j * _CHUNK, _CHUNK)], ssem.at[b]
        )

    def wait_scatter(b):
        pltpu.make_async_copy(
            emb_hbm.at[pl.ds(0, _CHUNK)], rows_v.at[b], ssem.at[b]
        ).wait()

    for b in range(_NBUF):
        start_gather(b, b)

    def outer(go, carry):
        for b in range(_NBUF):
            g = go * _NBUF + b
            wait_gather(b)
            start_scatter(g, b)
            wait_scatter(b)

            @pl.when(g + _NBUF < _NCHUNKS)
            def _():
                start_gather(g + _NBUF, b)

        return carry

    lax.fori_loop(0, _NCHUNKS // _NBUF, outer, 0)


_sc_gather = functools.partial(
    pl.kernel,
    mesh=plsc.VectorSubcoreMesh(core_axis_name="c", subcore_axis_name="s"),
    out_type=jax.ShapeDtypeStruct((_NA, _EMB), jnp.float32),
    scratch_types=[
        pltpu.VMEM((_NCHUNKS, _CHUNK), jnp.int32),
        pltpu.VMEM((_NBUF, _CHUNK, _EMB), jnp.float32),
        pltpu.SemaphoreType.DMA((_NBUF,)),
        pltpu.SemaphoreType.DMA((_NBUF,)),
    ],
)(_sc_gather_body)


def kernel(t_sampled, batch):
    emb = _emb_table(t_sampled.astype(jnp.float32))
    batch_r = batch.reshape(_NW, _NCHUNKS, _CHUNK)
    return _sc_gather(emb, batch_r)


# speedup vs baseline: 13.9544x; 3.5650x over previous
import functools

import jax
import jax.numpy as jnp
from jax import lax
from jax.experimental import pallas as pl
from jax.experimental.pallas import tpu as pltpu
from jax.experimental.pallas import tpu_sc as plsc

_EMB = 128
_HALF = 64
_NG = 8192
_NA = 524288

_NC = 2
_NS = 16
_NW = _NC * _NS

_CHUNK = 128
_ROWS_PER_W = _NA // _NW
_NCHUNKS = _ROWS_PER_W // _CHUNK


def _emb_body(t_ref, out_ref):
    t = t_ref[:, :]
    k = lax.broadcasted_iota(jnp.int32, (1, _EMB), 1).astype(jnp.float32)
    kmod = jnp.where(k < _HALF, k, k - _HALF)
    freqs = jnp.exp(-jnp.log(10000.0) * kmod / (_HALF - 1))
    phase = jnp.where(k < _HALF, 0.0, jnp.pi / 2.0)
    out_ref[:, :] = jnp.sin(t * freqs + phase)


def _emb_table(t_sampled):
    return pl.pallas_call(
        _emb_body,
        out_shape=jax.ShapeDtypeStruct((_NG, _EMB), jnp.float32),
    )(t_sampled)


_NBUF = 2
assert _NCHUNKS % _NBUF == 0


def _sc_gather_body(emb_hbm, batch_hbm, out_hbm, tbl_sh, idx_v, rows_v, gsem, ssem):
    sid = lax.axis_index("s")
    wid = sid * _NC + lax.axis_index("c")
    base = wid * _ROWS_PER_W

    @pl.when(sid == 0)
    def _():
        pltpu.sync_copy(emb_hbm, tbl_sh)

    pltpu.sync_copy(batch_hbm.at[wid], idx_v)
    plsc.subcore_barrier()

    def start_gather(j, b):
        pltpu.async_copy(tbl_sh.at[idx_v.at[j]], rows_v.at[b], gsem.at[b])

    def wait_gather(b):
        pltpu.make_async_copy(
            emb_hbm.at[pl.ds(0, _CHUNK)], rows_v.at[b], gsem.at[b]
        ).wait()

    def start_scatter(j, b):
        pltpu.async_copy(
            rows_v.at[b], out_hbm.at[pl.ds(base + j * _CHUNK, _CHUNK)], ssem.at[b]
        )

    def wait_scatter(b):
        pltpu.make_async_copy(
            emb_hbm.at[pl.ds(0, _CHUNK)], rows_v.at[b], ssem.at[b]
        ).wait()

    for b in range(_NBUF):
        start_gather(b, b)

    def outer(go, carry):
        for b in range(_NBUF):
            g = go * _NBUF + b
            wait_gather(b)
            start_scatter(g, b)
            wait_scatter(b)

            @pl.when(g + _NBUF < _NCHUNKS)
            def _():
                start_gather(g + _NBUF, b)

        return carry

    lax.fori_loop(0, _NCHUNKS // _NBUF, outer, 0)


_sc_gather = functools.partial(
    pl.kernel,
    mesh=plsc.VectorSubcoreMesh(core_axis_name="c", subcore_axis_name="s"),
    out_type=jax.ShapeDtypeStruct((_NA, _EMB), jnp.float32),
    scratch_types=[
        pltpu.VMEM_SHARED((_NG, _EMB), jnp.float32),
        pltpu.VMEM((_NCHUNKS, _CHUNK), jnp.int32),
        pltpu.VMEM((_NBUF, _CHUNK, _EMB), jnp.float32),
        pltpu.SemaphoreType.DMA((_NBUF,)),
        pltpu.SemaphoreType.DMA((_NBUF,)),
    ],
)(_sc_gather_body)


def kernel(t_sampled, batch):
    emb = _emb_table(t_sampled.astype(jnp.float32))
    batch_r = batch.reshape(_NW, _NCHUNKS, _CHUNK)
    return _sc_gather(emb, batch_r)


# speedup vs baseline: 14.1647x; 1.0151x over previous
import functools

import jax
import jax.numpy as jnp
from jax import lax
from jax.experimental import pallas as pl
from jax.experimental.pallas import tpu as pltpu
from jax.experimental.pallas import tpu_sc as plsc

_EMB = 128
_HALF = 64
_NG = 8192
_NA = 524288

_NC = 2
_NS = 16
_NW = _NC * _NS

_CHUNK = 64
_ROWS_PER_W = _NA // _NW
_NCHUNKS = _ROWS_PER_W // _CHUNK
_LEAD = 2


def _emb_body(t_ref, out_ref):
    t = t_ref[:, :]
    k = lax.broadcasted_iota(jnp.int32, (1, _EMB), 1).astype(jnp.float32)
    kmod = jnp.where(k < _HALF, k, k - _HALF)
    freqs = jnp.exp(-jnp.log(10000.0) * kmod / (_HALF - 1))
    phase = jnp.where(k < _HALF, 0.0, jnp.pi / 2.0)
    out_ref[:, :] = jnp.sin(t * freqs + phase)


def _emb_table(t_sampled):
    return pl.pallas_call(
        _emb_body,
        out_shape=jax.ShapeDtypeStruct((_NG, _EMB), jnp.float32),
    )(t_sampled)


_NBUF = 4
assert _NCHUNKS % _NBUF == 0
assert 0 < _LEAD < _NBUF


def _sc_gather_body(emb_hbm, batch_hbm, out_hbm, tbl_sh, idx_v, rows_v, gsem, ssem):
    sid = lax.axis_index("s")
    wid = sid * _NC + lax.axis_index("c")
    base = wid * _ROWS_PER_W

    @pl.when(sid == 0)
    def _():
        pltpu.sync_copy(emb_hbm, tbl_sh)

    pltpu.sync_copy(batch_hbm.at[wid], idx_v)
    plsc.subcore_barrier()

    def start_gather(j, b):
        pltpu.async_copy(tbl_sh.at[idx_v.at[j]], rows_v.at[b], gsem.at[b])

    def wait_gather(b):
        pltpu.make_async_copy(
            emb_hbm.at[pl.ds(0, _CHUNK)], rows_v.at[b], gsem.at[b]
        ).wait()

    def start_scatter(j, b):
        pltpu.async_copy(
            rows_v.at[b], out_hbm.at[pl.ds(base + j * _CHUNK, _CHUNK)], ssem.at[b]
        )

    def wait_scatter(b):
        pltpu.make_async_copy(
            emb_hbm.at[pl.ds(0, _CHUNK)], rows_v.at[b], ssem.at[b]
        ).wait()

    for j in range(_LEAD):
        start_gather(j, j)

    n_outer = _NCHUNKS // _NBUF

    def outer(go, carry):
        for b in range(_NBUF):
            g = go * _NBUF + b
            wait_gather(b)
            start_scatter(g, b)
            bg = (b + _LEAD) % _NBUF

            if b >= _NBUF - _LEAD:
                wait_scatter(bg)
            else:

                @pl.when(go > 0)
                def _():
                    wait_scatter(bg)

            if b < _NBUF - _LEAD:
                start_gather(g + _LEAD, bg)
            else:

                @pl.when(go < n_outer - 1)
                def _():
                    start_gather(g + _LEAD, bg)

        return carry

    lax.fori_loop(0, n_outer, outer, 0)

    for k in range(_NBUF - _LEAD):
        wait_scatter((_NCHUNKS - _NBUF + _LEAD + k) % _NBUF)


_sc_gather = functools.partial(
    pl.kernel,
    mesh=plsc.VectorSubcoreMesh(core_axis_name="c", subcore_axis_name="s"),
    out_type=jax.ShapeDtypeStruct((_NA, _EMB), jnp.float32),
    scratch_types=[
        pltpu.VMEM_SHARED((_NG, _EMB), jnp.float32),
        pltpu.VMEM((_NCHUNKS, _CHUNK), jnp.int32),
        pltpu.VMEM((_NBUF, _CHUNK, _EMB), jnp.float32),
        pltpu.SemaphoreType.DMA((_NBUF,)),
        pltpu.SemaphoreType.DMA((_NBUF,)),
    ],
)(_sc_gather_body)


def kernel(t_sampled, batch):
    emb = _emb_table(t_sampled.astype(jnp.float32))
    batch_r = batch.reshape(_NW, _NCHUNKS, _CHUNK)
    return _sc_gather(emb, batch_r)
